# Initial kernel scaffold; baseline (speedup 1.0000x reference)
import functools

import jax
import jax.numpy as jnp
from jax import lax
from jax.experimental import pallas as pl
from jax.experimental.pallas import tpu as pltpu
from jax.experimental.pallas import tpu_sc as plsc

NC = 2
NS = 16
NW = NC * NS
CHUNK = 128


def _pre_body(x_ref, wt_ref, b_ref, a_ref, ab_ref, h_ref, s_ref):
    h = jnp.dot(x_ref[...], wt_ref[...], preferred_element_type=jnp.float32)
    h = h + b_ref[...]
    h_ref[...] = h
    s_ref[...] = jnp.dot(h, a_ref[...], preferred_element_type=jnp.float32) + ab_ref[...]


def _tc_pre(hidden, wt, b_row, a_pad, ab_row):
    n, din = hidden.shape
    dout = wt.shape[1]
    blk = 1000
    grid = n // blk
    return pl.pallas_call(
        _pre_body,
        grid=(grid,),
        in_specs=[
            pl.BlockSpec((blk, din), lambda i: (i, 0)),
            pl.BlockSpec((din, dout), lambda i: (0, 0)),
            pl.BlockSpec((1, dout), lambda i: (0, 0)),
            pl.BlockSpec((dout, dout), lambda i: (0, 0)),
            pl.BlockSpec((1, dout), lambda i: (0, 0)),
        ],
        out_specs=[
            pl.BlockSpec((blk, dout), lambda i: (i, 0)),
            pl.BlockSpec((blk, dout), lambda i: (i, 0)),
        ],
        out_shape=[
            jax.ShapeDtypeStruct((n, dout), jnp.float32),
            jax.ShapeDtypeStruct((n, dout), jnp.float32),
        ],
    )(hidden, wt, b_row, a_pad, ab_row)


def _post_body(p0_ref, p1_ref, d0_ref, d1_ref, o_ref):
    den = d0_ref[...] + d1_ref[...]
    num = p0_ref[...] + p1_ref[...]
    o_ref[...] = jnp.where(den > 0.0, num / jnp.where(den > 0.0, den, 1.0), 0.0)


def _tc_post(p0, p1, d0, d1):
    n, d = p0.shape
    blk = 1000
    grid = n // blk
    return pl.pallas_call(
        _post_body,
        grid=(grid,),
        in_specs=[
            pl.BlockSpec((blk, d), lambda i: (i, 0)),
            pl.BlockSpec((blk, d), lambda i: (i, 0)),
            pl.BlockSpec((blk, 1), lambda i: (i, 0)),
            pl.BlockSpec((blk, 1), lambda i: (i, 0)),
        ],
        out_specs=pl.BlockSpec((blk, d), lambda i: (i, 0)),
        out_shape=jax.ShapeDtypeStruct((n, d), jnp.float32),
    )(p0, p1, d0, d1)


def _build_sc(n, d, ch, per):
    nden = ((n + NW - 1) // NW) * NW
    rows_per_tile = n // NS
    den_per_tile = nden // NS
    assert n % NS == 0 and rows_per_tile % 5 == 0
    row_step = rows_per_tile // 5

    mesh = plsc.VectorSubcoreMesh(core_axis_name="c", subcore_axis_name="s")

    @functools.partial(
        pl.kernel,
        out_type=[
            jax.ShapeDtypeStruct((NC, n, d), jnp.float32),
            jax.ShapeDtypeStruct((NC, nden), jnp.float32),
        ],
        mesh=mesh,
        scratch_types=[
            pltpu.VMEM((ch, CHUNK), jnp.int32),
            pltpu.VMEM((ch, CHUNK), jnp.int32),
            pltpu.VMEM((ch, CHUNK), jnp.float32),
            pltpu.VMEM((n,), jnp.float32),
            pltpu.VMEM((n,), jnp.float32),
            pltpu.VMEM((CHUNK, d), jnp.float32),
            pltpu.VMEM((nden // NS,), jnp.float32),
            pltpu.VMEM_SHARED((n, d), jnp.float32),
            pltpu.VMEM_SHARED((nden,), jnp.float32),
            pltpu.SemaphoreType.DMA,
        ],
    )
    def sc(src_hbm, dst_hbm, s1_hbm, s2_hbm, h_hbm, outp_hbm, den_hbm,
           src_v, dst_v, ex_v, s1_v, s2_v, rows_v, zden_v, acc_s, den_s, sem):
        cid = lax.axis_index("c")
        sid = lax.axis_index("s")
        wid = sid * NC + cid
        zeros16 = jnp.zeros((16,), jnp.float32)
        lane = lax.iota(jnp.int32, 16)

        def zrow(r, _):
            for k in range(d // 16):
                rows_v[r, pl.ds(k * 16, 16)] = zeros16
            return 0
        lax.fori_loop(0, CHUNK, zrow, 0)

        def zden(i, _):
            zden_v[pl.ds(i * 16, 16)] = zeros16
            return 0
        lax.fori_loop(0, den_per_tile // 16, zden, 0)

        for c5 in range(5):
            pltpu.sync_copy(
                rows_v.at[pl.ds(0, row_step)],
                acc_s.at[pl.ds(sid * rows_per_tile + c5 * row_step, row_step)])
        pltpu.sync_copy(zden_v, den_s.at[pl.ds(sid * den_per_tile, den_per_tile)])
        plsc.subcore_barrier()

        pltpu.sync_copy(src_hbm.at[wid], src_v)
        pltpu.sync_copy(dst_hbm.at[wid], dst_v)
        pltpu.sync_copy(s1_hbm, s1_v)
        pltpu.sync_copy(s2_hbm, s2_v)

        def pha(j, _):
            for k in range(CHUNK // 16):
                sv = src_v[j, pl.ds(k * 16, 16)]
                dv = dst_v[j, pl.ds(k * 16, 16)]
                e = plsc.load_gather(s1_v, [sv]) + plsc.load_gather(s2_v, [dv])
                e = jnp.where(e >= 0.0, e, e * jnp.float32(0.01))
                ex = jnp.exp(e)
                valid = (j * CHUNK + (k * 16) + lane) < per
                ex_v[j, pl.ds(k * 16, 16)] = jnp.where(valid, ex, 0.0)
            return 0
        lax.fori_loop(0, ch, pha, 0)

        def phb(j, _):
            pltpu.async_copy(h_hbm.at[src_v.at[j]], rows_v, sem).wait()

            def scale_row(r, _):
                a = ex_v[j, r]
                for k in range(d // 16):
                    rows_v[r, pl.ds(k * 16, 16)] = rows_v[r, pl.ds(k * 16, 16)] * a
                return 0
            lax.fori_loop(0, CHUNK, scale_row, 0)
            pltpu.sync_copy(rows_v, acc_s.at[dst_v.at[j]], add=True)
            pltpu.sync_copy(ex_v.at[j], den_s.at[dst_v.at[j]], add=True)
            return 0
        lax.fori_loop(0, ch, phb, 0)
        plsc.subcore_barrier()

        for c5 in range(5):
            b0 = sid * rows_per_tile + c5 * row_step
            pltpu.sync_copy(acc_s.at[pl.ds(b0, row_step)],
                            outp_hbm.at[cid, pl.ds(b0, row_step)])
        pltpu.sync_copy(den_s.at[pl.ds(sid * den_per_tile, den_per_tile)],
                        den_hbm.at[cid, pl.ds(sid * den_per_tile, den_per_tile)])

    return sc, nden


def kernel(hidden, edge_index, W_lin, b_lin, W_att, b_att):
    n, din = hidden.shape
    dout = W_lin.shape[0]
    e_total = edge_index.shape[1]

    a_pad = jnp.zeros((dout, dout), jnp.float32)
    a_pad = a_pad.at[:, 0].set(W_att[0, :dout]).at[:, 1].set(W_att[0, dout:])
    ab_row = jnp.zeros((1, dout), jnp.float32).at[0, 1].set(b_att[0])
    h, s = _tc_pre(hidden, W_lin.T, b_lin.reshape(1, dout), a_pad, ab_row)
    s1 = s[:, 0]
    s2 = s[:, 1]

    per = e_total // NW
    ch = (per + CHUNK - 1) // CHUNK
    per_pad = ch * CHUNK
    src = edge_index[0].astype(jnp.int32).reshape(NW, per)
    dst = edge_index[1].astype(jnp.int32).reshape(NW, per)
    src_p = jnp.zeros((NW, per_pad), jnp.int32).at[:, :per].set(src)
    dst_p = jnp.zeros((NW, per_pad), jnp.int32).at[:, :per].set(dst)
    src_p = src_p.reshape(NW, ch, CHUNK)
    dst_p = dst_p.reshape(NW, ch, CHUNK)

    sc, nden = _build_sc(n, dout, ch, per)
    outp, denp = sc(src_p, dst_p, s1, s2, h)

    out = _tc_post(outp[0], outp[1],
                   denp[0, :n].reshape(n, 1), denp[1, :n].reshape(n, 1))
    return out


# speedup vs baseline: 15.5009x; 15.5009x over previous
import functools

import jax
import jax.numpy as jnp
from jax import lax
from jax.experimental import pallas as pl
from jax.experimental.pallas import tpu as pltpu
from jax.experimental.pallas import tpu_sc as plsc

NC = 2
NS = 16
NW = NC * NS
CHUNK = 128


def _pre_body(x_ref, wt_ref, b_ref, a_ref, ab_ref, h_ref, s_ref):
    h = jnp.dot(x_ref[...], wt_ref[...], preferred_element_type=jnp.float32)
    h = h + b_ref[...]
    h_ref[...] = h
    s_ref[...] = jnp.dot(h, a_ref[...], preferred_element_type=jnp.float32) + ab_ref[...]


def _tc_pre(hidden, wt, b_row, a_pad, ab_row):
    n, din = hidden.shape
    dout = wt.shape[1]
    blk = 1000
    grid = n // blk
    return pl.pallas_call(
        _pre_body,
        grid=(grid,),
        in_specs=[
            pl.BlockSpec((blk, din), lambda i: (i, 0)),
            pl.BlockSpec((din, dout), lambda i: (0, 0)),
            pl.BlockSpec((1, dout), lambda i: (0, 0)),
            pl.BlockSpec((dout, dout), lambda i: (0, 0)),
            pl.BlockSpec((1, dout), lambda i: (0, 0)),
        ],
        out_specs=[
            pl.BlockSpec((blk, dout), lambda i: (i, 0)),
            pl.BlockSpec((blk, dout), lambda i: (i, 0)),
        ],
        out_shape=[
            jax.ShapeDtypeStruct((n, dout), jnp.float32),
            jax.ShapeDtypeStruct((n, dout), jnp.float32),
        ],
    )(hidden, wt, b_row, a_pad, ab_row)


def _post_body(p0_ref, p1_ref, d0_ref, d1_ref, o_ref):
    den = d0_ref[...] + d1_ref[...]
    num = p0_ref[...] + p1_ref[...]
    o_ref[...] = jnp.where(den > 0.0, num / jnp.where(den > 0.0, den, 1.0), 0.0)


def _tc_post(p0, p1, d0, d1):
    n, d = p0.shape
    blk = 1000
    grid = n // blk
    return pl.pallas_call(
        _post_body,
        grid=(grid,),
        in_specs=[
            pl.BlockSpec((blk, d), lambda i: (i, 0)),
            pl.BlockSpec((blk, d), lambda i: (i, 0)),
            pl.BlockSpec((blk, 1), lambda i: (i, 0)),
            pl.BlockSpec((blk, 1), lambda i: (i, 0)),
        ],
        out_specs=pl.BlockSpec((blk, d), lambda i: (i, 0)),
        out_shape=jax.ShapeDtypeStruct((n, d), jnp.float32),
    )(p0, p1, d0, d1)


def _build_sc(n, d, ch, per):
    npad = ((n + NS * CHUNK - 1) // (NS * CHUNK)) * (NS * CHUNK)
    nden = npad
    rows_per_tile = npad // NS
    den_per_tile = nden // NS
    row_chunks = rows_per_tile // CHUNK
    row_step = CHUNK

    mesh = plsc.VectorSubcoreMesh(core_axis_name="c", subcore_axis_name="s")

    @functools.partial(
        pl.kernel,
        out_type=[
            jax.ShapeDtypeStruct((NC, npad, d), jnp.float32),
            jax.ShapeDtypeStruct((NC, nden), jnp.float32),
        ],
        mesh=mesh,
        compiler_params=pltpu.CompilerParams(needs_layout_passes=False),
        scratch_types=[
            pltpu.VMEM((CHUNK,), jnp.int32),
            pltpu.VMEM((CHUNK,), jnp.int32),
            pltpu.VMEM((CHUNK,), jnp.float32),
            pltpu.VMEM((n,), jnp.float32),
            pltpu.VMEM((n,), jnp.float32),
            pltpu.VMEM((CHUNK, d), jnp.float32),
            pltpu.VMEM((nden // NS,), jnp.float32),
            pltpu.VMEM_SHARED((npad, d), jnp.float32),
            pltpu.VMEM_SHARED((nden,), jnp.float32),
            pltpu.SemaphoreType.DMA,
        ],
    )
    def sc(src_hbm, dst_hbm, s1_hbm, s2_hbm, h_hbm, outp_hbm, den_hbm,
           sidx_v, didx_v, exc_v, s1_v, s2_v, rows_v, zden_v, acc_s, den_s, sem):
        cid = lax.axis_index("c")
        sid = lax.axis_index("s")
        wid = sid * NC + cid
        zeros16 = jnp.zeros((16,), jnp.float32)
        lane = lax.iota(jnp.int32, 16)

        def zrow(r, _):
            for k in range(d // 16):
                rows_v[r, pl.ds(k * 16, 16)] = zeros16
            return 0
        lax.fori_loop(0, CHUNK, zrow, 0)

        def zden(i, _):
            zden_v[pl.ds(i * 16, 16)] = zeros16
            return 0
        lax.fori_loop(0, den_per_tile // 16, zden, 0)

        for c5 in range(row_chunks):
            pltpu.sync_copy(
                rows_v.at[pl.ds(0, row_step)],
                acc_s.at[pl.ds(sid * rows_per_tile + c5 * row_step, row_step)])
        pltpu.sync_copy(zden_v, den_s.at[pl.ds(sid * den_per_tile, den_per_tile)])
        plsc.subcore_barrier()

        pltpu.sync_copy(s1_hbm, s1_v)
        pltpu.sync_copy(s2_hbm, s2_v)

        def body(j, _):
            pltpu.sync_copy(src_hbm.at[wid, j], sidx_v)
            pltpu.sync_copy(dst_hbm.at[wid, j], didx_v)
            cp = pltpu.async_copy(h_hbm.at[sidx_v], rows_v, sem)
            for k in range(CHUNK // 16):
                sv = sidx_v[pl.ds(k * 16, 16)]
                dv = didx_v[pl.ds(k * 16, 16)]
                e = plsc.load_gather(s1_v, [sv]) + plsc.load_gather(s2_v, [dv])
                e = jnp.where(e >= 0.0, e, e * jnp.float32(0.01))
                ex = jnp.exp(e)
                valid = (j * CHUNK + (k * 16) + lane) < per
                exc_v[pl.ds(k * 16, 16)] = jnp.where(valid, ex, 0.0)
            cp.wait()

            def scale_grp(g, _):
                exv = exc_v[pl.ds(g * 16, 16)]
                for i in range(16):
                    a = exv[i]
                    r = g * 16 + i
                    for k in range(d // 16):
                        rows_v[r, pl.ds(k * 16, 16)] = rows_v[r, pl.ds(k * 16, 16)] * a
                return 0
            lax.fori_loop(0, CHUNK // 16, scale_grp, 0)
            pltpu.sync_copy(rows_v, acc_s.at[didx_v], add=True)
            pltpu.sync_copy(exc_v, den_s.at[didx_v], add=True)
            return 0
        lax.fori_loop(0, ch, body, 0)
        plsc.subcore_barrier()

        for c5 in range(row_chunks):
            b0 = sid * rows_per_tile + c5 * row_step
            pltpu.sync_copy(acc_s.at[pl.ds(b0, row_step)],
                            outp_hbm.at[cid, pl.ds(b0, row_step)])
        pltpu.sync_copy(den_s.at[pl.ds(sid * den_per_tile, den_per_tile)],
                        den_hbm.at[cid, pl.ds(sid * den_per_tile, den_per_tile)])

    return sc, nden


def kernel(hidden, edge_index, W_lin, b_lin, W_att, b_att):
    n, din = hidden.shape
    dout = W_lin.shape[0]
    e_total = edge_index.shape[1]

    a_pad = jnp.zeros((dout, dout), jnp.float32)
    a_pad = a_pad.at[:, 0].set(W_att[0, :dout]).at[:, 1].set(W_att[0, dout:])
    ab_row = jnp.zeros((1, dout), jnp.float32).at[0, 1].set(b_att[0])
    h, s = _tc_pre(hidden, W_lin.T, b_lin.reshape(1, dout), a_pad, ab_row)
    s1 = s[:, 0]
    s2 = s[:, 1]

    per = e_total // NW
    ch = (per + CHUNK - 1) // CHUNK
    per_pad = ch * CHUNK
    src = edge_index[0].astype(jnp.int32).reshape(NW, per)
    dst = edge_index[1].astype(jnp.int32).reshape(NW, per)
    src_p = jnp.zeros((NW, per_pad), jnp.int32).at[:, :per].set(src)
    dst_p = jnp.zeros((NW, per_pad), jnp.int32).at[:, :per].set(dst)
    src_p = src_p.reshape(NW, ch, CHUNK)
    dst_p = dst_p.reshape(NW, ch, CHUNK)

    sc, nden = _build_sc(n, dout, ch, per)
    outp, denp = sc(src_p, dst_p, s1, s2, h)

    out = _tc_post(outp[0, :n], outp[1, :n],
                   denp[0, :n].reshape(n, 1), denp[1, :n].reshape(n, 1))
    return out
